# baseline (device time: 45053 ns/iter reference)
import jax
import jax.numpy as jnp
from jax import lax
from jax.experimental import pallas as pl
from jax.experimental.pallas import tpu as pltpu

N_DEV = 16
SQ = 256
D = 1024
HQ_PER = 8
DH = 128
SKV = 4096
SCALE = 0.08838834764831843

CH = SQ // N_DEV


def _fused(x2, Wq, K4, V4, Wo):
    def body(x_ref, wq_ref, k_hbm, v_hbm, wo_ref, out_ref,
             k_buf, v_buf, copy_sems, pbuf, r_ref, g_ref,
             rs_send, rs_recv, ag_send, ag_recv):
        me = lax.axis_index("i")

        barrier_sem = pltpu.get_barrier_semaphore()
        for d in range(1, N_DEV):
            pl.semaphore_signal(
                barrier_sem, inc=1,
                device_id=(lax.rem(me + d, N_DEV),),
                device_id_type=pl.DeviceIdType.MESH,
            )

        def kv_copies(h):
            slot = h % 3
            ck = pltpu.make_async_copy(
                k_hbm.at[0, :, h, :], k_buf.at[slot], copy_sems.at[slot, 0]
            )
            cv = pltpu.make_async_copy(
                v_hbm.at[0, :, h, :], v_buf.at[slot], copy_sems.at[slot, 1]
            )
            return ck, cv

        xb = x_ref[...].astype(jnp.bfloat16)
        wqb = wq_ref[...].astype(jnp.bfloat16)
        wob = wo_ref[...].astype(jnp.bfloat16)
        qb = (
            jnp.dot(xb, wqb, preferred_element_type=jnp.float32)
            * (SCALE * 1.4426950408889634)
        ).astype(jnp.bfloat16)
        ones8 = jnp.ones((SKV, 8), jnp.bfloat16)

        ck, cv = kv_copies(0)
        ck.start()
        cv.start()
        for h in range(HQ_PER):
            slot = h % 3
            if h + 1 < HQ_PER:
                nk, nv = kv_copies(h + 1)
                nk.start()
                nv.start()
            ck.wait()
            s = lax.dot_general(
                qb[:, h * DH:(h + 1) * DH],
                k_buf[slot].astype(jnp.bfloat16),
                (((1,), (1,)), ((), ())),
                preferred_element_type=jnp.float32,
            )
            p = jnp.exp2(s).astype(jnp.bfloat16)
            cv.wait()
            attn_un = jnp.dot(
                p, v_buf[slot].astype(jnp.bfloat16),
                preferred_element_type=jnp.float32,
            )
            l = jnp.dot(p, ones8, preferred_element_type=jnp.float32)
            attn = attn_un / l[:, 0:1]
            contrib = jnp.dot(
                attn.astype(jnp.bfloat16),
                wob[h * DH:(h + 1) * DH, :],
                preferred_element_type=jnp.float32,
            )
            if h == 0:
                out_ref[0] = contrib
            else:
                out_ref[0] += contrib
            if h + 1 < HQ_PER:
                ck, cv = nk, nv

        pbuf[...] = out_ref[0].astype(jnp.bfloat16)

        pl.semaphore_wait(barrier_sem, N_DEV - 1)

        rs_rdmas = []
        for d in range(1, N_DEV):
            t = lax.rem(me + d, N_DEV)
            rdma = pltpu.make_async_remote_copy(
                src_ref=pbuf.at[pl.ds(t * CH, CH), :],
                dst_ref=r_ref.at[N_DEV - d],
                send_sem=rs_send.at[d],
                recv_sem=rs_recv.at[N_DEV - d],
                device_id=(t,),
                device_id_type=pl.DeviceIdType.MESH,
            )
            rdma.start()
            rs_rdmas.append(rdma)
        for k in range(1, N_DEV):
            recv = pltpu.make_async_remote_copy(
                src_ref=r_ref.at[k],
                dst_ref=r_ref.at[k],
                send_sem=rs_send.at[k],
                recv_sem=rs_recv.at[k],
                device_id=(me,),
                device_id_type=pl.DeviceIdType.MESH,
            )
            recv.wait_recv()
        mine = pl.ds(me * CH, CH)
        out_ref[0, mine, :] += jnp.sum(
            r_ref[1:N_DEV].astype(jnp.float32), axis=0
        )
        g_ref[0] = out_ref[0, mine, :].astype(jnp.bfloat16)

        ag_rdmas = []
        for d in range(1, N_DEV):
            t = lax.rem(me + d, N_DEV)
            rdma = pltpu.make_async_remote_copy(
                src_ref=g_ref.at[0],
                dst_ref=g_ref.at[N_DEV - d],
                send_sem=ag_send.at[d],
                recv_sem=ag_recv.at[N_DEV - d],
                device_id=(t,),
                device_id_type=pl.DeviceIdType.MESH,
            )
            rdma.start()
            ag_rdmas.append(rdma)
        for k in range(1, N_DEV):
            recv = pltpu.make_async_remote_copy(
                src_ref=g_ref.at[k],
                dst_ref=g_ref.at[k],
                send_sem=ag_send.at[k],
                recv_sem=ag_recv.at[k],
                device_id=(me,),
                device_id_type=pl.DeviceIdType.MESH,
            )
            recv.wait_recv()
            rows = pl.ds(lax.rem(me + k, N_DEV) * CH, CH)
            out_ref[0, rows, :] = g_ref[k].astype(jnp.float32)

        for rdma in rs_rdmas:
            rdma.wait_send()
        for rdma in ag_rdmas:
            rdma.wait_send()

    return pl.pallas_call(
        body,
        out_shape=jax.ShapeDtypeStruct((1, SQ, D), jnp.float32),
        in_specs=[
            pl.BlockSpec(memory_space=pltpu.VMEM),
            pl.BlockSpec(memory_space=pltpu.VMEM),
            pl.BlockSpec(memory_space=pl.ANY),
            pl.BlockSpec(memory_space=pl.ANY),
            pl.BlockSpec(memory_space=pltpu.VMEM),
        ],
        out_specs=pl.BlockSpec(memory_space=pltpu.VMEM),
        scratch_shapes=[
            pltpu.VMEM((3, SKV, DH), jnp.float32),
            pltpu.VMEM((3, SKV, DH), jnp.float32),
            pltpu.SemaphoreType.DMA((3, 2)),
            pltpu.VMEM((SQ, D), jnp.bfloat16),
            pltpu.VMEM((N_DEV, CH, D), jnp.bfloat16),
            pltpu.VMEM((N_DEV, CH, D), jnp.bfloat16),
            pltpu.SemaphoreType.DMA((N_DEV,)),
            pltpu.SemaphoreType.DMA((N_DEV,)),
            pltpu.SemaphoreType.DMA((N_DEV,)),
            pltpu.SemaphoreType.DMA((N_DEV,)),
        ],
        compiler_params=pltpu.CompilerParams(collective_id=0),
    )(x2, Wq, K4, V4, Wo)


def kernel(x, Wq, Wo, K_ext, V_ext):
    x2 = x.reshape(SQ, D)
    return _fused(x2, Wq, K_ext, V_ext, Wo)
